# baseline (speedup 1.0000x reference)
import functools

import jax
import jax.numpy as jnp
from jax.experimental import pallas as pl


_SELU_SCALE = 1.0507009873554804934193349852946
_SELU_ALPHA = 1.6732632423543772848170429916717
_TSPLIT = 8


def _body(*refs, n_time):
    a_refs, (x_ref, w_ref, o_ref) = refs[:_TSPLIT], refs[_TSPLIT:]
    j = pl.program_id(1)
    w = w_ref[...].astype(jnp.bfloat16)
    hs = []
    for k in range(_TSPLIT):
        a = a_refs[k][0].astype(jnp.bfloat16)
        x = x_ref[j * _TSPLIT + k].astype(jnp.bfloat16)
        h = jnp.dot(a, x, preferred_element_type=jnp.float32)
        h = jnp.dot(h.astype(jnp.bfloat16), w, preferred_element_type=jnp.float32)
        h = _SELU_SCALE * jnp.where(h > 0, h, _SELU_ALPHA * (jnp.exp(h) - 1.0))
        hs.append(h)
    o_ref[...] = jnp.stack(hs, axis=1)


@functools.partial(jax.jit, static_argnames=("block_n",))
def _run(Feature_tensor, adj, W, block_n=128):
    n_time, n_node, d = Feature_tensor.shape
    grid = (n_node // block_n, n_time // _TSPLIT)
    adj_specs = [
        pl.BlockSpec((1, block_n, n_node),
                     functools.partial(lambda k, i, j: (_TSPLIT * j + k, i, 0), k))
        for k in range(_TSPLIT)
    ]
    return pl.pallas_call(
        functools.partial(_body, n_time=n_time),
        grid=grid,
        in_specs=adj_specs + [
            pl.BlockSpec((n_time, n_node, d), lambda i, j: (0, 0, 0)),
            pl.BlockSpec((d, d), lambda i, j: (0, 0)),
        ],
        out_specs=pl.BlockSpec((block_n, _TSPLIT, d), lambda i, j: (i, j, 0)),
        out_shape=jax.ShapeDtypeStruct((n_node, n_time, d), jnp.float32),
    )(*([adj] * _TSPLIT), Feature_tensor, W)


def kernel(Feature_tensor, adj, W):
    return _run(Feature_tensor, adj, W)


# speedup vs baseline: 1.1386x; 1.1386x over previous
import functools

import jax
import jax.numpy as jnp
from jax.experimental import pallas as pl
from jax.experimental.pallas import tpu as pltpu


_SELU_SCALE = 1.0507009873554804934193349852946
_SELU_ALPHA = 1.6732632423543772848170429916717
_TSPLIT = 8


def _body(*refs):
    a_refs, (x_ref, w_ref, o_ref, y_ref) = refs[:_TSPLIT], refs[_TSPLIT:]
    i = pl.program_id(1)

    @pl.when(i == 0)
    def _precompute():
        w = w_ref[...].astype(jnp.bfloat16)
        for k in range(_TSPLIT):
            y = jnp.dot(x_ref[k].astype(jnp.bfloat16), w,
                        preferred_element_type=jnp.float32)
            y_ref[k] = y.astype(jnp.bfloat16)

    hs = []
    for k in range(_TSPLIT):
        a = a_refs[k][0].astype(jnp.bfloat16)
        h = jnp.dot(a, y_ref[k], preferred_element_type=jnp.float32)
        h = _SELU_SCALE * jnp.where(h > 0, h, _SELU_ALPHA * (jnp.exp(h) - 1.0))
        hs.append(h)
    o_ref[...] = jnp.stack(hs, axis=1)


@functools.partial(jax.jit, static_argnames=("block_n",))
def _run(Feature_tensor, adj, W, block_n=256):
    n_time, n_node, d = Feature_tensor.shape
    grid = (n_time // _TSPLIT, n_node // block_n)
    adj_specs = [
        pl.BlockSpec((1, block_n, n_node),
                     functools.partial(lambda k, j, i: (_TSPLIT * j + k, i, 0), k))
        for k in range(_TSPLIT)
    ]
    return pl.pallas_call(
        _body,
        grid=grid,
        in_specs=adj_specs + [
            pl.BlockSpec((_TSPLIT, n_node, d), lambda j, i: (j, 0, 0)),
            pl.BlockSpec((d, d), lambda j, i: (0, 0)),
        ],
        out_specs=pl.BlockSpec((block_n, _TSPLIT, d), lambda j, i: (i, j, 0)),
        out_shape=jax.ShapeDtypeStruct((n_node, n_time, d), jnp.float32),
        scratch_shapes=[pltpu.VMEM((_TSPLIT, n_node, d), jnp.bfloat16)],
    )(*([adj] * _TSPLIT), Feature_tensor, W)


def kernel(Feature_tensor, adj, W):
    return _run(Feature_tensor, adj, W)


# speedup vs baseline: 1.1398x; 1.0010x over previous
import functools

import jax
import jax.numpy as jnp
from jax.experimental import pallas as pl
from jax.experimental.pallas import tpu as pltpu


_SELU_SCALE = 1.0507009873554804934193349852946
_SELU_ALPHA = 1.6732632423543772848170429916717
_TSPLIT = 8


def _body(*refs):
    a_refs, (x_ref, w_ref, o_ref, y_ref) = refs[:_TSPLIT], refs[_TSPLIT:]
    i = pl.program_id(1)

    @pl.when(i == 0)
    def _precompute():
        w = w_ref[...].astype(jnp.bfloat16)
        for k in range(_TSPLIT):
            y = jnp.dot(x_ref[k].astype(jnp.bfloat16), w,
                        preferred_element_type=jnp.float32)
            y_ref[k] = y.astype(jnp.bfloat16)

    hs = []
    for k in range(_TSPLIT):
        a = a_refs[k][0].astype(jnp.bfloat16)
        h = jnp.dot(a, y_ref[k], preferred_element_type=jnp.float32)
        h = _SELU_SCALE * jnp.where(h > 0, h, _SELU_ALPHA * (jnp.exp(h) - 1.0))
        hs.append(h)
    o_ref[...] = jnp.stack(hs, axis=1)


@functools.partial(jax.jit, static_argnames=("block_n",))
def _run(Feature_tensor, adj, W, block_n=256):
    n_time, n_node, d = Feature_tensor.shape
    grid = (n_time // _TSPLIT, n_node // block_n)
    adj_specs = [
        pl.BlockSpec((1, block_n, n_node),
                     functools.partial(lambda k, j, i: (_TSPLIT * j + k, i, 0), k))
        for k in range(_TSPLIT)
    ]
    return pl.pallas_call(
        _body,
        grid=grid,
        in_specs=adj_specs + [
            pl.BlockSpec((_TSPLIT, n_node, d), lambda j, i: (j, 0, 0)),
            pl.BlockSpec((d, d), lambda j, i: (0, 0)),
        ],
        out_specs=pl.BlockSpec((block_n, _TSPLIT, d), lambda j, i: (i, j, 0)),
        out_shape=jax.ShapeDtypeStruct((n_node, n_time, d), jnp.float32),
        scratch_shapes=[pltpu.VMEM((_TSPLIT, n_node, d), jnp.bfloat16)],
        compiler_params=pltpu.CompilerParams(
            dimension_semantics=("parallel", "arbitrary")),
    )(*([adj] * _TSPLIT), Feature_tensor, W)


def kernel(Feature_tensor, adj, W):
    return _run(Feature_tensor, adj, W)


# speedup vs baseline: 1.1635x; 1.0208x over previous
import functools

import jax
import jax.numpy as jnp
from jax.experimental import pallas as pl
from jax.experimental.pallas import tpu as pltpu


_SELU_SCALE = 1.0507009873554804934193349852946
_SELU_ALPHA = 1.6732632423543772848170429916717
_TSPLIT = 8


def _body(*refs):
    a_refs, (x_ref, w_ref, o_ref, y_ref) = refs[:_TSPLIT // 2], refs[_TSPLIT // 2:]
    i = pl.program_id(1)

    @pl.when(i == 0)
    def _precompute():
        w = w_ref[...].astype(jnp.bfloat16)
        for k in range(_TSPLIT):
            y = jnp.dot(x_ref[k].astype(jnp.bfloat16), w,
                        preferred_element_type=jnp.float32)
            y_ref[k] = y.astype(jnp.bfloat16)

    hs = []
    for k in range(_TSPLIT):
        a = a_refs[k // 2][k % 2].astype(jnp.bfloat16)
        h = jnp.dot(a, y_ref[k], preferred_element_type=jnp.float32)
        h = _SELU_SCALE * jnp.where(h > 0, h, _SELU_ALPHA * (jnp.exp(h) - 1.0))
        hs.append(h)
    o_ref[...] = jnp.stack(hs, axis=1)


@functools.partial(jax.jit, static_argnames=("block_n",))
def _run(Feature_tensor, adj, W, block_n=256):
    n_time, n_node, d = Feature_tensor.shape
    grid = (n_time // _TSPLIT, n_node // block_n)
    adj_specs = [
        pl.BlockSpec((2, block_n, n_node),
                     functools.partial(lambda k, j, i: ((_TSPLIT * j) // 2 + k, i, 0), k))
        for k in range(_TSPLIT // 2)
    ]
    return pl.pallas_call(
        _body,
        grid=grid,
        in_specs=adj_specs + [
            pl.BlockSpec((_TSPLIT, n_node, d), lambda j, i: (j, 0, 0)),
            pl.BlockSpec((d, d), lambda j, i: (0, 0)),
        ],
        out_specs=pl.BlockSpec((block_n, _TSPLIT, d), lambda j, i: (i, j, 0)),
        out_shape=jax.ShapeDtypeStruct((n_node, n_time, d), jnp.float32),
        scratch_shapes=[pltpu.VMEM((_TSPLIT, n_node, d), jnp.bfloat16)],
        compiler_params=pltpu.CompilerParams(
            dimension_semantics=("parallel", "arbitrary")),
    )(*([adj] * (_TSPLIT // 2)), Feature_tensor, W)


def kernel(Feature_tensor, adj, W):
    return _run(Feature_tensor, adj, W)


# speedup vs baseline: 1.1665x; 1.0026x over previous
import functools

import jax
import jax.numpy as jnp
from jax.experimental import pallas as pl
from jax.experimental.pallas import tpu as pltpu


_SELU_SCALE = 1.0507009873554804934193349852946
_SELU_ALPHA = 1.6732632423543772848170429916717
_TSPLIT = 8


def _body(*refs):
    a_refs, (x_ref, w_ref, o_ref, y_ref) = refs[:_TSPLIT // 4], refs[_TSPLIT // 4:]
    i = pl.program_id(1)

    @pl.when(i == 0)
    def _precompute():
        w = w_ref[...].astype(jnp.bfloat16)
        for k in range(_TSPLIT):
            y = jnp.dot(x_ref[k].astype(jnp.bfloat16), w,
                        preferred_element_type=jnp.float32)
            y_ref[k] = y.astype(jnp.bfloat16)

    hs = []
    for k in range(_TSPLIT):
        a = a_refs[k // 4][k % 4].astype(jnp.bfloat16)
        h = jnp.dot(a, y_ref[k], preferred_element_type=jnp.float32)
        h = _SELU_SCALE * jnp.where(h > 0, h, _SELU_ALPHA * (jnp.exp(h) - 1.0))
        hs.append(h)
    o_ref[...] = jnp.stack(hs, axis=1)


@functools.partial(jax.jit, static_argnames=("block_n",))
def _run(Feature_tensor, adj, W, block_n=256):
    n_time, n_node, d = Feature_tensor.shape
    grid = (n_time // _TSPLIT, n_node // block_n)
    adj_specs = [
        pl.BlockSpec((4, block_n, n_node),
                     functools.partial(lambda k, j, i: ((_TSPLIT * j) // 4 + k, i, 0), k))
        for k in range(_TSPLIT // 4)
    ]
    return pl.pallas_call(
        _body,
        grid=grid,
        in_specs=adj_specs + [
            pl.BlockSpec((_TSPLIT, n_node, d), lambda j, i: (j, 0, 0)),
            pl.BlockSpec((d, d), lambda j, i: (0, 0)),
        ],
        out_specs=pl.BlockSpec((block_n, _TSPLIT, d), lambda j, i: (i, j, 0)),
        out_shape=jax.ShapeDtypeStruct((n_node, n_time, d), jnp.float32),
        scratch_shapes=[pltpu.VMEM((_TSPLIT, n_node, d), jnp.bfloat16)],
        compiler_params=pltpu.CompilerParams(
            dimension_semantics=("parallel", "arbitrary")),
    )(*([adj] * (_TSPLIT // 4)), Feature_tensor, W)


def kernel(Feature_tensor, adj, W):
    return _run(Feature_tensor, adj, W)
